# baseline (speedup 1.0000x reference)
import functools

import jax
import jax.numpy as jnp
from jax.experimental import pallas as pl
from jax.experimental.pallas import tpu as pltpu

_Q = 1024
_D = 16
_KBLK = 4000
_BIGF = float(2 ** 25)


def _tie_threshold(m):
    s = jnp.sqrt(m)
    base = jax.lax.bitcast_convert_type(s * s, jnp.int32)
    t = m
    for off in range(-2, 8):
        c = jax.lax.bitcast_convert_type(jnp.maximum(base + off, 0), jnp.float32)
        t = jnp.where(jnp.sqrt(c) == s, jnp.maximum(t, c), t)
    return t


def _dist_block_t(xt, pb):
    dot = jax.lax.dot_general(
        pb, xt, (((1,), (0,)), ((), ())),
        preferred_element_type=jnp.float32,
        precision=jax.lax.Precision.DEFAULT,
    )
    x2 = jnp.sum(xt * xt, axis=0, keepdims=True)
    p2 = jnp.sum(pb * pb, axis=1, keepdims=True)
    return x2 + p2 - 2.0 * dot


def _nn_kernel(nsteps, xt_ref, pb_ref, out_ref, minv_ref, mini_ref, thr_ref):
    phase = pl.program_id(0)
    step = pl.program_id(1)
    v = _dist_block_t(xt_ref[...], pb_ref[...])

    @pl.when(phase == 0)
    def _():
        bmin = jnp.min(v, axis=0, keepdims=True)

        @pl.when(step == 0)
        def _():
            minv_ref[...] = bmin

        @pl.when(step > 0)
        def _():
            minv_ref[...] = jnp.minimum(minv_ref[...], bmin)

        @pl.when(step == nsteps - 1)
        def _():
            thr_ref[...] = _tie_threshold(jnp.maximum(minv_ref[...], 0.0))

    @pl.when(phase == 1)
    def _():
        lane = jax.lax.broadcasted_iota(
            jnp.int32, (v.shape[0], 1), 0).astype(jnp.float32)
        hit = jnp.where(v <= thr_ref[...], lane, _BIGF)
        bidx = jnp.min(hit, axis=0, keepdims=True) + step * float(_KBLK)

        @pl.when(step == 0)
        def _():
            mini_ref[...] = bidx

        @pl.when(step > 0)
        def _():
            mini_ref[...] = jnp.minimum(mini_ref[...], bidx)

        @pl.when(step == nsteps - 1)
        def _():
            out_ref[...] = mini_ref[...].astype(jnp.int32)


def kernel(X, Phi):
    k = Phi.shape[0]
    nsteps = k // _KBLK
    assert nsteps * _KBLK == k
    xt = X.T

    out = pl.pallas_call(
        functools.partial(_nn_kernel, nsteps),
        grid=(2, nsteps),
        in_specs=[
            pl.BlockSpec((_D, _Q), lambda p, j: (0, 0)),
            pl.BlockSpec((_KBLK, _D), lambda p, j: (j, 0)),
        ],
        out_specs=pl.BlockSpec((1, _Q), lambda p, j: (0, 0)),
        out_shape=jax.ShapeDtypeStruct((1, _Q), jnp.int32),
        scratch_shapes=[
            pltpu.VMEM((1, _Q), jnp.float32),
            pltpu.VMEM((1, _Q), jnp.float32),
            pltpu.VMEM((1, _Q), jnp.float32),
        ],
    )(xt, Phi)
    return out.reshape(-1)


# speedup vs baseline: 1.0472x; 1.0472x over previous
import functools

import jax
import jax.numpy as jnp
from jax.experimental import pallas as pl
from jax.experimental.pallas import tpu as pltpu

_Q = 1024
_D = 16
_KBLK = 8192
_BIGF = float(2 ** 25)


def _dist_block(x, pt):
    dot = jax.lax.dot_general(
        x, pt, (((1,), (0,)), ((), ())),
        preferred_element_type=jnp.float32,
        precision=jax.lax.Precision.DEFAULT,
    )
    x2 = jnp.sum(x * x, axis=1, keepdims=True)
    p2 = jnp.sum(pt * pt, axis=0, keepdims=True)
    return x2 + p2 - 2.0 * dot


def _first_hit(v, t):
    lane = jax.lax.broadcasted_iota(
        jnp.int32, (1, v.shape[1]), 1).astype(jnp.float32)
    hit = jnp.where(v <= t, lane, _BIGF)
    return jnp.min(hit, axis=1, keepdims=True)


def _tie_threshold(m):
    s = jnp.sqrt(m)
    base = jax.lax.bitcast_convert_type(s * s, jnp.int32)
    t = m
    for off in range(-2, 8):
        c = jax.lax.bitcast_convert_type(jnp.maximum(base + off, 0), jnp.float32)
        t = jnp.where(jnp.sqrt(c) == s, jnp.maximum(t, c), t)
    return t


def _nn_kernel(nsteps, x_ref, pt_ref, out_ref, minv_ref, mini_ref, thr_ref):
    phase = pl.program_id(0)
    step = pl.program_id(1)
    v = _dist_block(x_ref[...], pt_ref[...])

    @pl.when(phase == 0)
    def _():
        bmin = jnp.min(v, axis=1, keepdims=True)

        @pl.when(step == 0)
        def _():
            minv_ref[...] = bmin

        @pl.when(step > 0)
        def _():
            minv_ref[...] = jnp.minimum(minv_ref[...], bmin)

        @pl.when(step == nsteps - 1)
        def _():
            thr_ref[...] = _tie_threshold(jnp.maximum(minv_ref[...], 0.0))

    @pl.when(phase == 1)
    def _():
        bidx = _first_hit(v, thr_ref[...]) + step * float(_KBLK)

        @pl.when(step == 0)
        def _():
            mini_ref[...] = bidx

        @pl.when(step > 0)
        def _():
            mini_ref[...] = jnp.minimum(mini_ref[...], bidx)

        @pl.when(step == nsteps - 1)
        def _():
            out_ref[...] = mini_ref[...].astype(jnp.int32)


def kernel(X, Phi):
    k = Phi.shape[0]
    nsteps = -(-k // _KBLK)
    kpad = nsteps * _KBLK
    phi_t = jnp.pad(Phi.T, ((0, 0), (0, kpad - k)), constant_values=1e17)

    out = pl.pallas_call(
        functools.partial(_nn_kernel, nsteps),
        grid=(2, nsteps),
        in_specs=[
            pl.BlockSpec((_Q, _D), lambda p, j: (0, 0)),
            pl.BlockSpec((_D, _KBLK), lambda p, j: (0, j)),
        ],
        out_specs=pl.BlockSpec((_Q, 1), lambda p, j: (0, 0)),
        out_shape=jax.ShapeDtypeStruct((_Q, 1), jnp.int32),
        scratch_shapes=[
            pltpu.VMEM((_Q, 1), jnp.float32),
            pltpu.VMEM((_Q, 1), jnp.float32),
            pltpu.VMEM((_Q, 1), jnp.float32),
        ],
    )(X, phi_t)
    return out.reshape(-1)
